# Initial kernel scaffold; baseline (speedup 1.0000x reference)
import jax
import jax.numpy as jnp
from jax.experimental import pallas as pl


def kernel(x, batch_idx, y, ln_gamma, ln_beta, W, b):
    raise NotImplementedError("write your pallas kernel here")


# speedup vs baseline: 5.6968x; 5.6968x over previous
import functools

import jax
import jax.numpy as jnp
from jax import lax
from jax.experimental import pallas as pl
from jax.experimental.pallas import tpu as pltpu
from jax.experimental.pallas import tpu_sc as plsc

N = 100000
D = 128
G = 512
RB = 2000
NSTEP = N // RB

NW = 32
P = 3136
NPAD = NW * P
CH = P // 16


def _rowval_body(x_ref, g_ref, bt_ref, wt_ref, o_ref):
    xb = x_ref[...]
    u = g_ref[...] * wt_ref[...]
    c = jnp.sum(bt_ref[...] * wt_ref[...])
    mean = jnp.mean(xb, axis=1, keepdims=True)
    d = xb - mean
    var = jnp.mean(d * d, axis=1, keepdims=True)
    sd = lax.dot_general(d, u, dimension_numbers=(((1,), (1,)), ((), ())),
                         preferred_element_type=jnp.float32)
    o_ref[...] = sd * lax.rsqrt(var + 1e-5) + c


_rowval_call = pl.pallas_call(
    _rowval_body,
    grid=(NSTEP,),
    in_specs=[
        pl.BlockSpec((RB, D), lambda i: (i, 0)),
        pl.BlockSpec((1, D), lambda i: (0, 0)),
        pl.BlockSpec((1, D), lambda i: (0, 0)),
        pl.BlockSpec((1, D), lambda i: (0, 0)),
    ],
    out_specs=pl.BlockSpec((RB, 1), lambda i: (i, 0)),
    out_shape=jax.ShapeDtypeStruct((N, 1), jnp.float32),
)


def _sc_seg_body(rv_hbm, idx_hbm, sums_out, cnts_out, rv_v, idx_v, sums_v, cnts_v):
    wid = lax.axis_index("s") * 2 + lax.axis_index("c")
    base = wid * P
    pltpu.sync_copy(rv_hbm.at[pl.ds(base, P)], rv_v)
    pltpu.sync_copy(idx_hbm.at[pl.ds(base, P)], idx_v)

    z16 = jnp.zeros((16,), jnp.float32)

    def zero_body(j, carry):
        sums_v[pl.ds(j * 16, 16)] = z16
        cnts_v[pl.ds(j * 16, 16)] = z16
        return carry

    lax.fori_loop(0, G // 16, zero_body, 0)

    lane = lax.iota(jnp.int32, 16)
    one16 = jnp.ones((16,), jnp.float32)

    def body(k, carry):
        off = k * 16
        rvv = rv_v[pl.ds(off, 16)]
        sg = idx_v[pl.ds(off, 16)]
        plsc.addupdate_scatter(sums_v, [sg], rvv)
        valid = (base + off + lane) < N
        plsc.addupdate_scatter(cnts_v, [sg], jnp.where(valid, one16, z16))
        return carry

    lax.fori_loop(0, CH, body, 0)

    pltpu.sync_copy(sums_v, sums_out.at[wid])
    pltpu.sync_copy(cnts_v, cnts_out.at[wid])


def _sc_seg_call():
    return pl.kernel(
        _sc_seg_body,
        out_type=(jax.ShapeDtypeStruct((NW, G), jnp.float32),
                  jax.ShapeDtypeStruct((NW, G), jnp.float32)),
        mesh=plsc.VectorSubcoreMesh(core_axis_name="c", subcore_axis_name="s"),
        compiler_params=pltpu.CompilerParams(needs_layout_passes=False),
        scratch_types=[
            pltpu.VMEM((P,), jnp.float32),
            pltpu.VMEM((P,), jnp.int32),
            pltpu.VMEM((G,), jnp.float32),
            pltpu.VMEM((G,), jnp.float32),
        ],
    )


def _epilogue_body(s_ref, c_ref, b_ref, o_ref):
    s = jnp.sum(s_ref[...], axis=0, keepdims=True)
    cc = jnp.sum(c_ref[...], axis=0, keepdims=True)
    o_ref[...] = s / jnp.maximum(cc, 1.0) + b_ref[...]


_epilogue_call = pl.pallas_call(
    _epilogue_body,
    out_shape=jax.ShapeDtypeStruct((1, G), jnp.float32),
)


@jax.jit
def kernel(x, batch_idx, y, ln_gamma, ln_beta, W, b):
    idx32 = batch_idx.astype(jnp.int32)
    g2 = ln_gamma.reshape(1, D)
    bt2 = ln_beta.reshape(1, D)
    wt2 = W.reshape(1, D)

    rv = _rowval_call(x, g2, bt2, wt2)
    rv_p = jnp.pad(rv.reshape(-1), (0, NPAD - N))
    idx_p = jnp.pad(idx32, (0, NPAD - N))

    sums, cnts = _sc_seg_call()(rv_p, idx_p)

    bb = jnp.broadcast_to(b.reshape(1, 1), (1, G))
    pred2 = _epilogue_call(sums, cnts, bb)
    return (pred2.reshape(G, 1), y)
